# Initial kernel scaffold; baseline (speedup 1.0000x reference)
import jax
import jax.numpy as jnp
from jax.experimental import pallas as pl


def kernel(dec_x, dec_pc, enc_x, enc_pc, W1, b1, W2, b2):
    raise NotImplementedError("write your pallas kernel here")


# speedup vs baseline: 16.5367x; 16.5367x over previous
import functools

import jax
import jax.numpy as jnp
from jax import lax
from jax.experimental import pallas as pl
from jax.experimental.pallas import tpu as pltpu
from jax.experimental.pallas import tpu_sc as plsc

_pc = pl.pallas_call

_B, _N, _M, _K = 8, 4096, 1024, 16
_IN, _D, _C = 128, 256, 3 + 128
_NC, _NS = 2, 16
_NW = _NC * _NS
_CH = 128
_TU = 512
_TMK = 256
_TME = 128


def _fps_body(px_ref, py_ref, pz_ref, idx_ref, fx_ref, fy_ref, fz_ref):
    px = px_ref[...]
    py = py_ref[...]
    pz = pz_ref[...]
    iota_n = lax.broadcasted_iota(jnp.int32, (_B, _N), 1)
    iota_m = lax.broadcasted_iota(jnp.int32, (_B, _M), 1)
    zero = jnp.zeros((_B, _N), dtype=jnp.float32)

    def body(i, st):
        dists, far, idxs, fx, fy, fz = st
        onehot = iota_n == far
        cx = jnp.sum(jnp.where(onehot, px, zero), axis=1, keepdims=True)
        cy = jnp.sum(jnp.where(onehot, py, zero), axis=1, keepdims=True)
        cz = jnp.sum(jnp.where(onehot, pz, zero), axis=1, keepdims=True)
        sel = iota_m == i
        zm_i = jnp.zeros((_B, _M), dtype=jnp.int32)
        zm_f = jnp.zeros((_B, _M), dtype=jnp.float32)
        idxs = jnp.where(sel, far + zm_i, idxs)
        fx = jnp.where(sel, cx + zm_f, fx)
        fy = jnp.where(sel, cy + zm_f, fy)
        fz = jnp.where(sel, cz + zm_f, fz)
        dx = px - cx
        dy = py - cy
        dz = pz - cz
        d = dx * dx + dy * dy + dz * dz
        dists = jnp.minimum(dists, d)
        mx = jnp.max(dists, axis=1, keepdims=True)
        far = jnp.min(jnp.where(dists == mx, iota_n, _N), axis=1, keepdims=True)
        return dists, far, idxs, fx, fy, fz

    dists0 = px * 0.0 + jnp.float32(1e10)
    far0 = (px[:, :1] * 0.0).astype(jnp.int32)
    f0 = px[:, :_M] * 0.0
    idxs0 = f0.astype(jnp.int32)
    _, _, idxs, fx, fy, fz = lax.fori_loop(
        0, _M, body, (dists0, far0, idxs0, f0, f0, f0))
    idx_ref[...] = idxs
    fx_ref[...] = fx
    fy_ref[...] = fy
    fz_ref[...] = fz


def _fps(dec_pc):
    px = dec_pc[:, :, 0]
    py = dec_pc[:, :, 1]
    pz = dec_pc[:, :, 2]
    out_shape = [
        jax.ShapeDtypeStruct((_B, _M), jnp.int32),
        jax.ShapeDtypeStruct((_B, _M), jnp.float32),
        jax.ShapeDtypeStruct((_B, _M), jnp.float32),
        jax.ShapeDtypeStruct((_B, _M), jnp.float32),
    ]
    idxs, fx, fy, fz = _pc(_fps_body, out_shape=out_shape)(px, py, pz)
    return jnp.stack([fx, fy, fz], axis=-1)


def _u_body(g_ref, w_ref, b_ref, u_ref):
    u_ref[...] = (
        jnp.dot(g_ref[...], w_ref[...], preferred_element_type=jnp.float32)
        + b_ref[...])


def _u_table(dec_pc, dec_x, W1, b1):
    g = jnp.concatenate([dec_pc, dec_x], axis=-1).reshape(_B * _N, _C)
    return _pc(
        _u_body,
        grid=(_B * _N // _TU,),
        in_specs=[
            pl.BlockSpec((_TU, _C), lambda i: (i, 0)),
            pl.BlockSpec((_C, _D), lambda i: (0, 0)),
            pl.BlockSpec((1, _D), lambda i: (0, 0)),
        ],
        out_specs=pl.BlockSpec((_TU, _D), lambda i: (i, 0)),
        out_shape=jax.ShapeDtypeStruct((_B * _N, _D), jnp.float32),
    )(g, W1, b1.reshape(1, _D))


def _knn_body(q_ref, pcT_ref, knn_ref):
    b = pl.program_id(0)
    q = q_ref[0]
    p = pcT_ref[0]
    px = p[0:1, :]
    py = p[1:2, :]
    pz = p[2:3, :]
    qx = q[:, 0:1]
    qy = q[:, 1:2]
    qz = q[:, 2:3]
    d2 = px * px + py * py + pz * pz
    d1 = qx * qx + qy * qy + qz * qz
    r = lambda v: v.astype(jnp.bfloat16).astype(jnp.float32)
    inner = (r(qx) * r(px) + r(qy) * r(py)) + r(qz) * r(pz)
    mat = (d1 + d2) - 2.0 * inner
    iota_n = lax.broadcasted_iota(jnp.int32, (_TMK, _N), 1)
    inf = jnp.float32(jnp.inf)
    cols = []
    for _ in range(_K):
        mn = jnp.min(mat, axis=1, keepdims=True)
        idx = jnp.min(jnp.where(mat == mn, iota_n, _N), axis=1, keepdims=True)
        cols.append(idx + b * _N)
        mat = jnp.where(iota_n == idx, inf, mat)
    knn_ref[0] = jnp.concatenate(cols, axis=1).astype(jnp.int32)


def _knn(pc_fps, dec_pc):
    pcT = jnp.transpose(dec_pc, (0, 2, 1))
    return _pc(
        _knn_body,
        grid=(_B, _M // _TMK),
        in_specs=[
            pl.BlockSpec((1, _TMK, 3), lambda b, m: (b, m, 0)),
            pl.BlockSpec((1, 3, _N), lambda b, m: (b, 0, 0)),
        ],
        out_specs=pl.BlockSpec((1, _TMK, _K), lambda b, m: (b, m, 0)),
        out_shape=jax.ShapeDtypeStruct((_B, _M, _K), jnp.int32),
    )(pc_fps, pcT)


def _sc_gather(u_flat, flat_idx):
    rows = flat_idx.shape[0]
    rpw = rows // _NW
    nch = rpw // _CH
    mesh = plsc.VectorSubcoreMesh(core_axis_name="c", subcore_axis_name="s")

    @functools.partial(
        pl.kernel,
        mesh=mesh,
        out_type=jax.ShapeDtypeStruct((rows, _D), jnp.float32),
        scratch_types=[
            pltpu.VMEM((_CH,), jnp.int32),
            pltpu.VMEM((_CH, _D), jnp.float32),
            pltpu.SemaphoreType.DMA,
        ],
    )
    def k(u_hbm, idx_hbm, out_hbm, idx_v, rows_v, sem):
        wid = lax.axis_index("s") * _NC + lax.axis_index("c")
        base = wid * rpw

        def chunk(c, carry):
            off = base + c * _CH
            pltpu.sync_copy(idx_hbm.at[pl.ds(off, _CH)], idx_v)
            pltpu.async_copy(u_hbm.at[idx_v], rows_v, sem).wait()
            pltpu.sync_copy(rows_v, out_hbm.at[pl.ds(off, _CH)])
            return carry

        lax.fori_loop(0, nch, chunk, 0)

    return k(u_flat, flat_idx)


def _mlp_body(h_ref, q_ref, w1p_ref, w2_ref, b2_ref, out_ref):
    q = q_ref[0]
    w1p = w1p_ref[...]
    qf = (q[:, 0:1] * w1p[0:1, :] + q[:, 1:2] * w1p[1:2, :]
          + q[:, 2:3] * w1p[2:3, :])
    h = h_ref[0]
    h3 = h.reshape(_TME, _K, _D) - qf[:, None, :]
    h3 = jnp.maximum(h3, 0.0)
    h2 = jnp.dot(h3.reshape(_TME * _K, _D), w2_ref[...],
                 preferred_element_type=jnp.float32)
    out_ref[0] = jnp.max(h2.reshape(_TME, _K, _D), axis=1) + b2_ref[...]


def _mlp(H, pc_fps, W1p, W2, b2):
    return _pc(
        _mlp_body,
        grid=(_B, _M // _TME),
        in_specs=[
            pl.BlockSpec((1, _TME * _K, _D), lambda b, m: (b, m, 0)),
            pl.BlockSpec((1, _TME, 3), lambda b, m: (b, m, 0)),
            pl.BlockSpec((3, _D), lambda b, m: (0, 0)),
            pl.BlockSpec((_D, _D), lambda b, m: (0, 0)),
            pl.BlockSpec((1, _D), lambda b, m: (0, 0)),
        ],
        out_specs=pl.BlockSpec((1, _TME, _D), lambda b, m: (b, m, 0)),
        out_shape=jax.ShapeDtypeStruct((_B, _M, _D), jnp.float32),
    )(H.reshape(_B, _M * _K, _D), pc_fps, W1p, W2, b2.reshape(1, _D))


def kernel(dec_x, dec_pc, enc_x, enc_pc, W1, b1, W2, b2):
    pc_fps = _fps(dec_pc)
    U = _u_table(dec_pc, dec_x, W1, b1)
    knn = _knn(pc_fps, dec_pc)
    H = _sc_gather(U, knn.reshape(_B * _M * _K))
    out = _mlp(H, pc_fps, W1[:3], W2, b2)
    return (out, pc_fps, enc_x, enc_pc)


# speedup vs baseline: 18.9591x; 1.1465x over previous
import functools

import jax
import jax.numpy as jnp
from jax import lax
from jax.experimental import pallas as pl
from jax.experimental.pallas import tpu as pltpu
from jax.experimental.pallas import tpu_sc as plsc

_pc = pl.pallas_call

_B, _N, _M, _K = 8, 4096, 1024, 16
_IN, _D, _C = 128, 256, 3 + 128
_NC, _NS = 2, 16
_NW = _NC * _NS
_CH = 128
_TU = 512
_TMK = 256
_TME = 128


def _fps_body(px_ref, py_ref, pz_ref, idx_ref, fx_ref, fy_ref, fz_ref):
    px = px_ref[...]
    py = py_ref[...]
    pz = pz_ref[...]
    iota_nf = lax.broadcasted_iota(jnp.int32, (_B, _N), 1).astype(jnp.float32)
    iota_mf = lax.broadcasted_iota(jnp.int32, (_B, _M), 1).astype(jnp.float32)
    zero = jnp.zeros((_B, _N), dtype=jnp.float32)
    nf = jnp.float32(_N)

    def body(i, st):
        dists, farf, idxs, fx, fy, fz = st
        onehot = iota_nf == farf
        cx = jnp.sum(jnp.where(onehot, px, zero), axis=1, keepdims=True)
        cy = jnp.sum(jnp.where(onehot, py, zero), axis=1, keepdims=True)
        cz = jnp.sum(jnp.where(onehot, pz, zero), axis=1, keepdims=True)
        sel = iota_mf == i.astype(jnp.float32)
        zm_f = jnp.zeros((_B, _M), dtype=jnp.float32)
        idxs = jnp.where(sel, farf + zm_f, idxs)
        fx = jnp.where(sel, cx + zm_f, fx)
        fy = jnp.where(sel, cy + zm_f, fy)
        fz = jnp.where(sel, cz + zm_f, fz)
        dx = px - cx
        dy = py - cy
        dz = pz - cz
        d = dx * dx + dy * dy + dz * dz
        dists = jnp.minimum(dists, d)
        mx = jnp.max(dists, axis=1, keepdims=True)
        farf = jnp.min(jnp.where(dists == mx, iota_nf, nf), axis=1,
                       keepdims=True)
        return dists, farf, idxs, fx, fy, fz

    dists0 = px * 0.0 + jnp.float32(1e10)
    far0 = px[:, :1] * 0.0
    f0 = px[:, :_M] * 0.0
    _, _, idxs, fx, fy, fz = lax.fori_loop(
        0, _M, body, (dists0, far0, f0, f0, f0, f0))
    idx_ref[...] = idxs.astype(jnp.int32)
    fx_ref[...] = fx
    fy_ref[...] = fy
    fz_ref[...] = fz


def _fps(dec_pc):
    px = dec_pc[:, :, 0]
    py = dec_pc[:, :, 1]
    pz = dec_pc[:, :, 2]
    out_shape = [
        jax.ShapeDtypeStruct((_B, _M), jnp.int32),
        jax.ShapeDtypeStruct((_B, _M), jnp.float32),
        jax.ShapeDtypeStruct((_B, _M), jnp.float32),
        jax.ShapeDtypeStruct((_B, _M), jnp.float32),
    ]
    idxs, fx, fy, fz = _pc(_fps_body, out_shape=out_shape)(px, py, pz)
    return jnp.stack([fx, fy, fz], axis=-1)


def _u_body(g_ref, w_ref, b_ref, u_ref):
    u_ref[...] = (
        jnp.dot(g_ref[...], w_ref[...], preferred_element_type=jnp.float32)
        + b_ref[...])


def _u_table(dec_pc, dec_x, W1, b1):
    g = jnp.concatenate([dec_pc, dec_x], axis=-1).reshape(_B * _N, _C)
    return _pc(
        _u_body,
        grid=(_B * _N // _TU,),
        in_specs=[
            pl.BlockSpec((_TU, _C), lambda i: (i, 0)),
            pl.BlockSpec((_C, _D), lambda i: (0, 0)),
            pl.BlockSpec((1, _D), lambda i: (0, 0)),
        ],
        out_specs=pl.BlockSpec((_TU, _D), lambda i: (i, 0)),
        out_shape=jax.ShapeDtypeStruct((_B * _N, _D), jnp.float32),
    )(g, W1, b1.reshape(1, _D))


def _knn_body(q_ref, pcT_ref, knn_ref):
    b = pl.program_id(0)
    q = q_ref[0]
    p = pcT_ref[0]
    px = p[0:1, :]
    py = p[1:2, :]
    pz = p[2:3, :]
    qx = q[:, 0:1]
    qy = q[:, 1:2]
    qz = q[:, 2:3]
    d2 = px * px + py * py + pz * pz
    d1 = qx * qx + qy * qy + qz * qz
    r = lambda v: v.astype(jnp.bfloat16).astype(jnp.float32)
    inner = (r(qx) * r(px) + r(qy) * r(py)) + r(qz) * r(pz)
    mat = (d1 + d2) - 2.0 * inner
    iota_f = lax.broadcasted_iota(jnp.int32, (_TMK, _N), 1).astype(jnp.float32)
    nf = jnp.float32(_N)
    inf = jnp.float32(jnp.inf)
    cols = []
    for _ in range(_K):
        mn = jnp.min(mat, axis=1, keepdims=True)
        cand = jnp.where(mat == mn, iota_f, nf)
        idxf = jnp.min(cand, axis=1, keepdims=True)
        cols.append(idxf)
        mat = jnp.where(cand == idxf, inf, mat)
    knn_ref[0] = jnp.concatenate(cols, axis=1).astype(jnp.int32) + b * _N


def _knn(pc_fps, dec_pc):
    pcT = jnp.transpose(dec_pc, (0, 2, 1))
    return _pc(
        _knn_body,
        grid=(_B, _M // _TMK),
        in_specs=[
            pl.BlockSpec((1, _TMK, 3), lambda b, m: (b, m, 0)),
            pl.BlockSpec((1, 3, _N), lambda b, m: (b, 0, 0)),
        ],
        out_specs=pl.BlockSpec((1, _TMK, _K), lambda b, m: (b, m, 0)),
        out_shape=jax.ShapeDtypeStruct((_B, _M, _K), jnp.int32),
    )(pc_fps, pcT)


def _sc_gather(u_flat, flat_idx):
    rows = flat_idx.shape[0]
    rpw = rows // _NW
    nch = rpw // _CH
    mesh = plsc.VectorSubcoreMesh(core_axis_name="c", subcore_axis_name="s")

    @functools.partial(
        pl.kernel,
        mesh=mesh,
        out_type=jax.ShapeDtypeStruct((rows, _D), jnp.float32),
        scratch_types=[
            pltpu.VMEM((_CH,), jnp.int32),
            pltpu.VMEM((_CH, _D), jnp.float32),
            pltpu.SemaphoreType.DMA,
        ],
    )
    def k(u_hbm, idx_hbm, out_hbm, idx_v, rows_v, sem):
        wid = lax.axis_index("s") * _NC + lax.axis_index("c")
        base = wid * rpw

        def chunk(c, carry):
            off = base + c * _CH
            pltpu.sync_copy(idx_hbm.at[pl.ds(off, _CH)], idx_v)
            pltpu.async_copy(u_hbm.at[idx_v], rows_v, sem).wait()
            pltpu.sync_copy(rows_v, out_hbm.at[pl.ds(off, _CH)])
            return carry

        lax.fori_loop(0, nch, chunk, 0)

    return k(u_flat, flat_idx)


def _mlp_body(h_ref, q_ref, w1p_ref, w2_ref, b2_ref, out_ref):
    q = q_ref[0]
    w1p = w1p_ref[...]
    qf = (q[:, 0:1] * w1p[0:1, :] + q[:, 1:2] * w1p[1:2, :]
          + q[:, 2:3] * w1p[2:3, :])
    h = h_ref[0]
    h3 = h.reshape(_TME, _K, _D) - qf[:, None, :]
    h3 = jnp.maximum(h3, 0.0)
    h2 = jnp.dot(h3.reshape(_TME * _K, _D), w2_ref[...],
                 preferred_element_type=jnp.float32)
    out_ref[0] = jnp.max(h2.reshape(_TME, _K, _D), axis=1) + b2_ref[...]


def _mlp(H, pc_fps, W1p, W2, b2):
    return _pc(
        _mlp_body,
        grid=(_B, _M // _TME),
        in_specs=[
            pl.BlockSpec((1, _TME * _K, _D), lambda b, m: (b, m, 0)),
            pl.BlockSpec((1, _TME, 3), lambda b, m: (b, m, 0)),
            pl.BlockSpec((3, _D), lambda b, m: (0, 0)),
            pl.BlockSpec((_D, _D), lambda b, m: (0, 0)),
            pl.BlockSpec((1, _D), lambda b, m: (0, 0)),
        ],
        out_specs=pl.BlockSpec((1, _TME, _D), lambda b, m: (b, m, 0)),
        out_shape=jax.ShapeDtypeStruct((_B, _M, _D), jnp.float32),
    )(H.reshape(_B, _M * _K, _D), pc_fps, W1p, W2, b2.reshape(1, _D))


def kernel(dec_x, dec_pc, enc_x, enc_pc, W1, b1, W2, b2):
    pc_fps = _fps(dec_pc)
    U = _u_table(dec_pc, dec_x, W1, b1)
    knn = _knn(pc_fps, dec_pc)
    H = _sc_gather(U, knn.reshape(_B * _M * _K))
    out = _mlp(H, pc_fps, W1[:3], W2, b2)
    return (out, pc_fps, enc_x, enc_pc)


# speedup vs baseline: 19.4781x; 1.0274x over previous
import functools

import jax
import jax.numpy as jnp
from jax import lax
from jax.experimental import pallas as pl
from jax.experimental.pallas import tpu as pltpu
from jax.experimental.pallas import tpu_sc as plsc

_pc = pl.pallas_call

_B, _N, _M, _K = 8, 4096, 1024, 16
_IN, _D, _C = 128, 256, 3 + 128
_NC, _NS = 2, 16
_NW = _NC * _NS
_CH = 128
_TU = 512
_TMK = 512
_TME = 128


def _fps_body(px_ref, py_ref, pz_ref, idx_ref, fx_ref, fy_ref, fz_ref):
    px = px_ref[...]
    py = py_ref[...]
    pz = pz_ref[...]
    iota_nf = lax.broadcasted_iota(jnp.int32, (_B, _N), 1).astype(jnp.float32)
    iota_mf = lax.broadcasted_iota(jnp.int32, (_B, _M), 1).astype(jnp.float32)
    zero = jnp.zeros((_B, _N), dtype=jnp.float32)
    nf = jnp.float32(_N)

    def body(i, st):
        dists, farf, idxs, fx, fy, fz = st
        onehot = iota_nf == farf
        cx = jnp.sum(jnp.where(onehot, px, zero), axis=1, keepdims=True)
        cy = jnp.sum(jnp.where(onehot, py, zero), axis=1, keepdims=True)
        cz = jnp.sum(jnp.where(onehot, pz, zero), axis=1, keepdims=True)
        sel = iota_mf == i.astype(jnp.float32)
        zm_f = jnp.zeros((_B, _M), dtype=jnp.float32)
        idxs = jnp.where(sel, farf + zm_f, idxs)
        fx = jnp.where(sel, cx + zm_f, fx)
        fy = jnp.where(sel, cy + zm_f, fy)
        fz = jnp.where(sel, cz + zm_f, fz)
        dx = px - cx
        dy = py - cy
        dz = pz - cz
        d = dx * dx + dy * dy + dz * dz
        dists = jnp.minimum(dists, d)
        mx = jnp.max(dists, axis=1, keepdims=True)
        farf = jnp.min(jnp.where(dists == mx, iota_nf, nf), axis=1,
                       keepdims=True)
        return dists, farf, idxs, fx, fy, fz

    dists0 = px * 0.0 + jnp.float32(1e10)
    far0 = px[:, :1] * 0.0
    f0 = px[:, :_M] * 0.0
    _, _, idxs, fx, fy, fz = lax.fori_loop(
        0, _M, body, (dists0, far0, f0, f0, f0, f0))
    idx_ref[...] = idxs.astype(jnp.int32)
    fx_ref[...] = fx
    fy_ref[...] = fy
    fz_ref[...] = fz


def _fps(dec_pc):
    px = dec_pc[:, :, 0]
    py = dec_pc[:, :, 1]
    pz = dec_pc[:, :, 2]
    out_shape = [
        jax.ShapeDtypeStruct((_B, _M), jnp.int32),
        jax.ShapeDtypeStruct((_B, _M), jnp.float32),
        jax.ShapeDtypeStruct((_B, _M), jnp.float32),
        jax.ShapeDtypeStruct((_B, _M), jnp.float32),
    ]
    idxs, fx, fy, fz = _pc(_fps_body, out_shape=out_shape)(px, py, pz)
    return jnp.stack([fx, fy, fz], axis=-1)


def _u_body(g_ref, w_ref, b_ref, u_ref):
    u_ref[...] = (
        jnp.dot(g_ref[...], w_ref[...], preferred_element_type=jnp.float32)
        + b_ref[...])


def _u_table(dec_pc, dec_x, W1, b1):
    g = jnp.concatenate([dec_pc, dec_x], axis=-1).reshape(_B * _N, _C)
    return _pc(
        _u_body,
        grid=(_B * _N // _TU,),
        in_specs=[
            pl.BlockSpec((_TU, _C), lambda i: (i, 0)),
            pl.BlockSpec((_C, _D), lambda i: (0, 0)),
            pl.BlockSpec((1, _D), lambda i: (0, 0)),
        ],
        out_specs=pl.BlockSpec((_TU, _D), lambda i: (i, 0)),
        out_shape=jax.ShapeDtypeStruct((_B * _N, _D), jnp.float32),
    )(g, W1, b1.reshape(1, _D))


def _knn_body(q_ref, pcT_ref, knn_ref):
    b = pl.program_id(0)
    q = q_ref[0]
    p = pcT_ref[0]
    px = p[0:1, :]
    py = p[1:2, :]
    pz = p[2:3, :]
    qx = q[:, 0:1]
    qy = q[:, 1:2]
    qz = q[:, 2:3]
    d2 = px * px + py * py + pz * pz
    d1 = qx * qx + qy * qy + qz * qz
    r = lambda v: v.astype(jnp.bfloat16).astype(jnp.float32)
    inner = (r(qx) * r(px) + r(qy) * r(py)) + r(qz) * r(pz)
    mat = (d1 + d2) - 2.0 * inner
    iota_f = lax.broadcasted_iota(jnp.int32, (_TMK, _N), 1).astype(jnp.float32)
    nf = jnp.float32(_N)
    inf = jnp.float32(jnp.inf)
    cols = []
    for _ in range(_K):
        mn = jnp.min(mat, axis=1, keepdims=True)
        cand = jnp.where(mat == mn, iota_f, nf)
        idxf = jnp.min(cand, axis=1, keepdims=True)
        cols.append(idxf)
        mat = jnp.where(cand == idxf, inf, mat)
    knn_ref[0] = jnp.concatenate(cols, axis=1).astype(jnp.int32) + b * _N


def _knn(pc_fps, dec_pc):
    pcT = jnp.transpose(dec_pc, (0, 2, 1))
    return _pc(
        _knn_body,
        grid=(_B, _M // _TMK),
        in_specs=[
            pl.BlockSpec((1, _TMK, 3), lambda b, m: (b, m, 0)),
            pl.BlockSpec((1, 3, _N), lambda b, m: (b, 0, 0)),
        ],
        out_specs=pl.BlockSpec((1, _TMK, _K), lambda b, m: (b, m, 0)),
        out_shape=jax.ShapeDtypeStruct((_B, _M, _K), jnp.int32),
    )(pc_fps, pcT)


def _sc_gather(u_flat, flat_idx):
    rows = flat_idx.shape[0]
    rpw = rows // _NW
    nch = rpw // _CH
    mesh = plsc.VectorSubcoreMesh(core_axis_name="c", subcore_axis_name="s")

    @functools.partial(
        pl.kernel,
        mesh=mesh,
        out_type=jax.ShapeDtypeStruct((rows, _D), jnp.float32),
        scratch_types=[
            pltpu.VMEM((_CH,), jnp.int32),
            pltpu.VMEM((_CH,), jnp.int32),
            pltpu.VMEM((_CH, _D), jnp.float32),
            pltpu.VMEM((_CH, _D), jnp.float32),
            pltpu.SemaphoreType.DMA,
            pltpu.SemaphoreType.DMA,
        ],
    )
    def k(u_hbm, idx_hbm, out_hbm, idx_v0, idx_v1, rows_v0, rows_v1,
          sem0, sem1):
        wid = lax.axis_index("s") * _NC + lax.axis_index("c")
        base = wid * rpw
        bufs = ((idx_v0, rows_v0, sem0), (idx_v1, rows_v1, sem1))

        pltpu.sync_copy(idx_hbm.at[pl.ds(base, _CH)], idx_v0)
        descs = [pltpu.async_copy(u_hbm.at[idx_v0], rows_v0, sem0)]
        for c in range(nch):
            idx_n, rows_n, sem_n = bufs[(c + 1) % 2]
            if c + 1 < nch:
                pltpu.sync_copy(
                    idx_hbm.at[pl.ds(base + (c + 1) * _CH, _CH)], idx_n)
                descs.append(pltpu.async_copy(u_hbm.at[idx_n], rows_n, sem_n))
            descs[c].wait()
            _, rows_c, _ = bufs[c % 2]
            pltpu.sync_copy(rows_c, out_hbm.at[pl.ds(base + c * _CH, _CH)])

    return k(u_flat, flat_idx)


def _mlp_body(h_ref, q_ref, w1p_ref, w2_ref, b2_ref, out_ref):
    q = q_ref[0]
    w1p = w1p_ref[...]
    qf = (q[:, 0:1] * w1p[0:1, :] + q[:, 1:2] * w1p[1:2, :]
          + q[:, 2:3] * w1p[2:3, :])
    h = h_ref[0]
    h3 = h.reshape(_TME, _K, _D) - qf[:, None, :]
    h3 = jnp.maximum(h3, 0.0)
    h2 = jnp.dot(h3.reshape(_TME * _K, _D), w2_ref[...],
                 preferred_element_type=jnp.float32)
    out_ref[0] = jnp.max(h2.reshape(_TME, _K, _D), axis=1) + b2_ref[...]


def _mlp(H, pc_fps, W1p, W2, b2):
    return _pc(
        _mlp_body,
        grid=(_B, _M // _TME),
        in_specs=[
            pl.BlockSpec((1, _TME * _K, _D), lambda b, m: (b, m, 0)),
            pl.BlockSpec((1, _TME, 3), lambda b, m: (b, m, 0)),
            pl.BlockSpec((3, _D), lambda b, m: (0, 0)),
            pl.BlockSpec((_D, _D), lambda b, m: (0, 0)),
            pl.BlockSpec((1, _D), lambda b, m: (0, 0)),
        ],
        out_specs=pl.BlockSpec((1, _TME, _D), lambda b, m: (b, m, 0)),
        out_shape=jax.ShapeDtypeStruct((_B, _M, _D), jnp.float32),
    )(H.reshape(_B, _M * _K, _D), pc_fps, W1p, W2, b2.reshape(1, _D))


def kernel(dec_x, dec_pc, enc_x, enc_pc, W1, b1, W2, b2):
    pc_fps = _fps(dec_pc)
    U = _u_table(dec_pc, dec_x, W1, b1)
    knn = _knn(pc_fps, dec_pc)
    H = _sc_gather(U, knn.reshape(_B * _M * _K))
    out = _mlp(H, pc_fps, W1[:3], W2, b2)
    return (out, pc_fps, enc_x, enc_pc)
